# baseline (speedup 1.0000x reference)
import functools

import jax
import jax.numpy as jnp
from jax import lax
from jax.experimental import pallas as pl
from jax.experimental.pallas import tpu as pltpu
from jax.experimental.pallas import tpu_sc as plsc

VOCAB = 1000000
D = 64
B_TOT = 16 * 2048
NW = 32
B_PER_W = B_TOT // NW
CHUNK = 128
NCHUNK = B_PER_W // CHUNK
SEQ = 2048


def _body(x_hbm, tok_hbm, pos_hbm, out_hbm, idx_v, rows_v, pos_v, sem):
    c = lax.axis_index("c")
    s = lax.axis_index("s")
    wid = s * 2 + c
    base = wid * B_PER_W
    s_base = lax.rem(base, SEQ)

    pltpu.sync_copy(x_hbm.at[pl.ds(wid * NCHUNK, NCHUNK)], idx_v)
    pltpu.sync_copy(pos_hbm.at[pl.ds(s_base, B_PER_W)], pos_v)

    for j in range(NCHUNK):
        pltpu.async_copy(tok_hbm.at[idx_v.at[j]], rows_v, sem).wait()

        def add_row(i, _, j=j):
            for t in range(D // 16):
                rows_v[i, pl.ds(t * 16, 16)] = (
                    rows_v[i, pl.ds(t * 16, 16)]
                    + pos_v[j * CHUNK + i, pl.ds(t * 16, 16)]
                )
            return 0

        lax.fori_loop(0, CHUNK, add_row, 0)

        pltpu.sync_copy(rows_v, out_hbm.at[pl.ds(base + j * CHUNK, CHUNK)])


@jax.jit
def _embed(x2d, token_table, pos_table):
    mesh = plsc.VectorSubcoreMesh(core_axis_name="c", subcore_axis_name="s")
    return pl.kernel(
        _body,
        out_type=jax.ShapeDtypeStruct((B_TOT, D), jnp.float32),
        mesh=mesh,
        scratch_types=[
            pltpu.VMEM((NCHUNK, CHUNK), jnp.int32),
            pltpu.VMEM((CHUNK, D), jnp.float32),
            pltpu.VMEM((B_PER_W, D), jnp.float32),
            pltpu.SemaphoreType.DMA,
        ],
        compiler_params=pltpu.CompilerParams(use_tc_tiling_on_sc=False),
    )(x2d, token_table, pos_table)


def kernel(x, token_table, pos_table):
    x2d = x.reshape(B_TOT // CHUNK, CHUNK).astype(jnp.int32)
    out = _embed(x2d, token_table, pos_table)
    return out.reshape(x.shape[0], x.shape[1], D)
